# baseline (speedup 1.0000x reference)
import functools

import jax
import jax.numpy as jnp
from jax import lax
from jax.experimental import pallas as pl
from jax.experimental.pallas import tpu as pltpu
from jax.experimental.pallas import tpu_sc as plsc

B = 1024
N = 100000
NPAD = 100096
K = 52
M = 26
NC = 2
NS = 16
NW = NC * NS
ROWS = B // NW
FLAT = ROWS * K
IDXW = 128
NIDX = FLAT // IDXW
CHUNK_OFFS = (0, 10, 26, 36)


@functools.cache
def _build_singlenet_sc():
    return functools.partial(
        pl.kernel,
        out_type=jax.ShapeDtypeStruct((B,), jnp.float32),
        mesh=plsc.VectorSubcoreMesh(core_axis_name="c", subcore_axis_name="s"),
        compiler_params=pltpu.CompilerParams(needs_layout_passes=False),
        scratch_types=[
            pltpu.VMEM((FLAT,), jnp.int32),
            pltpu.VMEM((FLAT,), jnp.float32),
            pltpu.VMEM((NPAD,), jnp.int32),
            pltpu.VMEM((FLAT,), jnp.float32),
            pltpu.VMEM((ROWS,), jnp.float32),
            pltpu.VMEM((16,), jnp.float32),
            [pltpu.SemaphoreType.DMA] * 2,
        ],
    )(_singlenet_sc)


def _singlenet_sc(edges_hbm, w_hbm, bias_hbm, out_hbm,
                  edges_v, wbuf, stamp, sibuf, outv, bias_v, sems):
    wid = lax.axis_index("s") * NC + lax.axis_index("c")
    base = wid * FLAT

    pltpu.sync_copy(edges_hbm.at[pl.ds(base, FLAT)], edges_v)

    gather = pltpu.async_copy(w_hbm.at[edges_v], wbuf, sems[0])
    bias_copy = pltpu.async_copy(bias_hbm, bias_v, sems[1])

    lanes = lax.iota(jnp.int32, 16)
    ids = [lanes + 16 * c for c in range(4)]
    signs = [jnp.full((16,), s, jnp.float32) for s in (1.0, 1.0, -1.0, -1.0)]
    zero = jnp.zeros((16,), jnp.float32)

    outv[pl.ds(0, 16)] = zero
    outv[pl.ds(16, 16)] = zero

    for i in range(ROWS):
        o = i * K
        e = [edges_v[pl.ds(o + off, 16)] for off in CHUNK_OFFS]
        for c in range(4):
            plsc.store_scatter(stamp, [e[c]], ids[c])
        for c, off in enumerate(CHUNK_OFFS):
            s = plsc.load_gather(stamp, [e[c]])
            sibuf[pl.ds(o + off, 16)] = jnp.where(s == ids[c], signs[c], zero)

    gather.wait()

    tail_f = jnp.where(lanes >= 12, 1.0, 0.0).astype(jnp.float32)
    for i in range(ROWS):
        o = i * K
        acc = sibuf[pl.ds(o, 16)] * wbuf[pl.ds(o, 16)]
        acc = acc + sibuf[pl.ds(o + 16, 16)] * wbuf[pl.ds(o + 16, 16)]
        acc = acc + sibuf[pl.ds(o + 32, 16)] * wbuf[pl.ds(o + 32, 16)]
        acc = acc + sibuf[pl.ds(o + 36, 16)] * wbuf[pl.ds(o + 36, 16)] * tail_f
        plsc.addupdate_scatter(outv, [lanes * 0 + i], acc)

    bias_copy.wait()
    b = bias_v[...]
    for c in range(ROWS // 16):
        x = outv[pl.ds(c * 16, 16)] + b
        outv[pl.ds(c * 16, 16)] = 1.0 / (1.0 + jnp.exp(-x))

    pltpu.sync_copy(outv, out_hbm.at[pl.ds(wid * ROWS, ROWS)])


@jax.jit
def kernel(edges, W, bias):
    edges_flat = edges.reshape(-1)
    w_flat = W.reshape(-1)
    bias16 = jnp.broadcast_to(bias, (16,))
    pred = _build_singlenet_sc()(edges_flat, w_flat, bias16)
    return pred.reshape(B, 1)


# speedup vs baseline: 1.0405x; 1.0405x over previous
import functools

import jax
import jax.numpy as jnp
from jax import lax
from jax.experimental import pallas as pl
from jax.experimental.pallas import tpu as pltpu
from jax.experimental.pallas import tpu_sc as plsc

B = 1024
N = 100000
NPAD = 100096
K = 52
M = 26
NC = 2
NS = 16
NW = NC * NS
ROWS = B // NW
FLAT = ROWS * K
IDXW = 128
NIDX = FLAT // IDXW
CHUNK_OFFS = (0, 10, 26, 36)


@functools.cache
def _build_singlenet_sc():
    return functools.partial(
        pl.kernel,
        out_type=jax.ShapeDtypeStruct((B,), jnp.float32),
        mesh=plsc.VectorSubcoreMesh(core_axis_name="c", subcore_axis_name="s"),
        compiler_params=pltpu.CompilerParams(needs_layout_passes=False),
        scratch_types=[
            pltpu.VMEM((FLAT,), jnp.int32),
            pltpu.VMEM((FLAT,), jnp.float32),
            pltpu.VMEM((NPAD,), jnp.int32),
            pltpu.VMEM((FLAT,), jnp.float32),
            pltpu.VMEM((ROWS,), jnp.float32),
            pltpu.VMEM((16,), jnp.float32),
            [pltpu.SemaphoreType.DMA] * 2,
        ],
    )(_singlenet_sc)


def _singlenet_sc(edges_hbm, w_hbm, bias_hbm, out_hbm,
                  edges_v, wbuf, stamp, sibuf, outv, bias_v, sems):
    wid = lax.axis_index("s") * NC + lax.axis_index("c")
    base = wid * FLAT

    pltpu.sync_copy(edges_hbm.at[pl.ds(base, FLAT)], edges_v)

    gather = pltpu.async_copy(w_hbm.at[edges_v], wbuf, sems[0])
    bias_copy = pltpu.async_copy(bias_hbm, bias_v, sems[1])

    lanes = lax.iota(jnp.int32, 16)
    ids = [lanes + 16 * c for c in range(4)]
    signs = [jnp.full((16,), s, jnp.float32) for s in (1.0, 1.0, -1.0, -1.0)]
    zero = jnp.zeros((16,), jnp.float32)

    outv[pl.ds(0, 16)] = zero
    outv[pl.ds(16, 16)] = zero

    def _stamp_row(i, _):
        o = i * K
        e = [edges_v[pl.ds(o + off, 16)] for off in CHUNK_OFFS]
        for c in range(4):
            plsc.store_scatter(stamp, [e[c]], ids[c])
        for c, off in enumerate(CHUNK_OFFS):
            s = plsc.load_gather(stamp, [e[c]])
            sibuf[pl.ds(o + off, 16)] = jnp.where(s == ids[c], signs[c], zero)
        return 0

    lax.fori_loop(0, ROWS, _stamp_row, 0)

    gather.wait()

    tail_f = jnp.where(lanes >= 12, 1.0, 0.0).astype(jnp.float32)

    def _product_row(i, _):
        o = i * K
        acc = sibuf[pl.ds(o, 16)] * wbuf[pl.ds(o, 16)]
        acc = acc + sibuf[pl.ds(o + 16, 16)] * wbuf[pl.ds(o + 16, 16)]
        acc = acc + sibuf[pl.ds(o + 32, 16)] * wbuf[pl.ds(o + 32, 16)]
        acc = acc + sibuf[pl.ds(o + 36, 16)] * wbuf[pl.ds(o + 36, 16)] * tail_f
        plsc.addupdate_scatter(outv, [lanes * 0 + i], acc)
        return 0

    lax.fori_loop(0, ROWS, _product_row, 0)

    bias_copy.wait()
    b = bias_v[...]
    for c in range(ROWS // 16):
        x = outv[pl.ds(c * 16, 16)] + b
        outv[pl.ds(c * 16, 16)] = 1.0 / (1.0 + jnp.exp(-x))

    pltpu.sync_copy(outv, out_hbm.at[pl.ds(wid * ROWS, ROWS)])


@jax.jit
def kernel(edges, W, bias):
    edges_flat = edges.reshape(-1)
    w_flat = W.reshape(-1)
    bias16 = jnp.broadcast_to(bias, (16,))
    pred = _build_singlenet_sc()(edges_flat, w_flat, bias16)
    return pred.reshape(B, 1)


# speedup vs baseline: 1.0472x; 1.0064x over previous
import functools

import jax
import jax.numpy as jnp
from jax import lax
from jax.experimental import pallas as pl
from jax.experimental.pallas import tpu as pltpu
from jax.experimental.pallas import tpu_sc as plsc

B = 1024
N = 100000
NPAD = 100096
K = 52
M = 26
NC = 2
NS = 16
NW = NC * NS
ROWS = B // NW
FLAT = ROWS * K
IDXW = 128
NIDX = FLAT // IDXW
CHUNK_OFFS = (0, 10, 26, 36)


@functools.cache
def _build_singlenet_sc():
    return functools.partial(
        pl.kernel,
        out_type=jax.ShapeDtypeStruct((B,), jnp.float32),
        mesh=plsc.VectorSubcoreMesh(core_axis_name="c", subcore_axis_name="s"),
        compiler_params=pltpu.CompilerParams(needs_layout_passes=False),
        scratch_types=[
            pltpu.VMEM((FLAT,), jnp.int32),
            pltpu.VMEM((FLAT,), jnp.float32),
            pltpu.VMEM((NPAD,), jnp.int32),
            pltpu.VMEM((FLAT,), jnp.float32),
            pltpu.VMEM((ROWS,), jnp.float32),
            pltpu.VMEM((16,), jnp.float32),
            [pltpu.SemaphoreType.DMA] * 2,
        ],
    )(_singlenet_sc)


def _singlenet_sc(edges_hbm, w_hbm, bias_hbm, out_hbm,
                  edges_v, wbuf, stamp, sibuf, outv, bias_v, sems):
    wid = lax.axis_index("s") * NC + lax.axis_index("c")
    base = wid * FLAT

    pltpu.sync_copy(edges_hbm.at[pl.ds(base, FLAT)], edges_v)

    gather = pltpu.async_copy(w_hbm.at[edges_v], wbuf, sems[0])
    bias_copy = pltpu.async_copy(bias_hbm, bias_v, sems[1])

    lanes = lax.iota(jnp.int32, 16)
    ids = [lanes + 16 * c for c in range(4)]
    signs = [jnp.full((16,), s, jnp.float32) for s in (1.0, 1.0, -1.0, -1.0)]
    zero = jnp.zeros((16,), jnp.float32)

    outv[pl.ds(0, 16)] = zero
    outv[pl.ds(16, 16)] = zero

    def _stamp_row(i, _):
        o = i * K
        e = [edges_v[pl.ds(o + off, 16)] for off in CHUNK_OFFS]
        for c in range(4):
            plsc.store_scatter(stamp, [e[c]], ids[c])
        for c, off in enumerate(CHUNK_OFFS):
            s = plsc.load_gather(stamp, [e[c]])
            sibuf[pl.ds(o + off, 16)] = jnp.where(s == ids[c], signs[c], zero)
        return 0

    lax.fori_loop(0, ROWS, _stamp_row, 0)

    gather.wait()

    tail_f = jnp.where(lanes >= 12, 1.0, 0.0).astype(jnp.float32)

    @plsc.parallel_loop(0, ROWS, unroll=2)
    def _product_row(i):
        o = i * K
        acc = sibuf[pl.ds(o, 16)] * wbuf[pl.ds(o, 16)]
        acc = acc + sibuf[pl.ds(o + 16, 16)] * wbuf[pl.ds(o + 16, 16)]
        acc = acc + sibuf[pl.ds(o + 32, 16)] * wbuf[pl.ds(o + 32, 16)]
        acc = acc + sibuf[pl.ds(o + 36, 16)] * wbuf[pl.ds(o + 36, 16)] * tail_f
        plsc.addupdate_scatter(outv, [lanes * 0 + i], acc)

    bias_copy.wait()
    b = bias_v[...]
    for c in range(ROWS // 16):
        x = outv[pl.ds(c * 16, 16)] + b
        outv[pl.ds(c * 16, 16)] = 1.0 / (1.0 + jnp.exp(-x))

    pltpu.sync_copy(outv, out_hbm.at[pl.ds(wid * ROWS, ROWS)])


@jax.jit
def kernel(edges, W, bias):
    edges_flat = edges.reshape(-1)
    w_flat = W.reshape(-1)
    bias16 = jnp.broadcast_to(bias, (16,))
    pred = _build_singlenet_sc()(edges_flat, w_flat, bias16)
    return pred.reshape(B, 1)


# speedup vs baseline: 1.0515x; 1.0041x over previous
import functools

import jax
import jax.numpy as jnp
from jax import lax
from jax.experimental import pallas as pl
from jax.experimental.pallas import tpu as pltpu
from jax.experimental.pallas import tpu_sc as plsc

B = 1024
N = 100000
NPAD = 100096
K = 52
M = 26
NC = 2
NS = 16
NW = NC * NS
ROWS = B // NW
FLAT = ROWS * K
IDXW = 128
NIDX = FLAT // IDXW
CHUNK_OFFS = (0, 10, 26, 36)


@functools.cache
def _build_singlenet_sc():
    return functools.partial(
        pl.kernel,
        out_type=jax.ShapeDtypeStruct((B,), jnp.float32),
        mesh=plsc.VectorSubcoreMesh(core_axis_name="c", subcore_axis_name="s"),
        compiler_params=pltpu.CompilerParams(needs_layout_passes=False),
        scratch_types=[
            pltpu.VMEM((FLAT,), jnp.int32),
            pltpu.VMEM((FLAT,), jnp.float32),
            pltpu.VMEM((NPAD,), jnp.int32),
            pltpu.VMEM((FLAT,), jnp.float32),
            pltpu.VMEM((ROWS,), jnp.float32),
            pltpu.VMEM((16,), jnp.float32),
            [pltpu.SemaphoreType.DMA] * 2,
        ],
    )(_singlenet_sc)


def _singlenet_sc(edges_hbm, w_hbm, bias_hbm, out_hbm,
                  edges_v, wbuf, stamp, sibuf, outv, bias_v, sems):
    wid = lax.axis_index("s") * NC + lax.axis_index("c")
    base = wid * FLAT

    pltpu.sync_copy(edges_hbm.at[pl.ds(base, FLAT)], edges_v)

    gather = pltpu.async_copy(w_hbm.at[edges_v], wbuf, sems[0])
    bias_copy = pltpu.async_copy(bias_hbm, bias_v, sems[1])

    lanes = lax.iota(jnp.int32, 16)
    ids = [lanes + 16 * c for c in range(4)]
    signs = [jnp.full((16,), s, jnp.float32) for s in (1.0, 1.0, -1.0, -1.0)]
    zero = jnp.zeros((16,), jnp.float32)

    outv[pl.ds(0, 16)] = zero
    outv[pl.ds(16, 16)] = zero

    def _stamp_row(i, _):
        o = i * K
        e = [edges_v[pl.ds(o + off, 16)] for off in CHUNK_OFFS]
        for c in range(4):
            plsc.store_scatter(stamp, [e[c]], ids[c])
        for c, off in enumerate(CHUNK_OFFS):
            s = plsc.load_gather(stamp, [e[c]])
            sibuf[pl.ds(o + off, 16)] = jnp.where(s == ids[c], signs[c], zero)
        return 0

    lax.fori_loop(0, ROWS, _stamp_row, 0, unroll=2)

    gather.wait()

    tail_f = jnp.where(lanes >= 12, 1.0, 0.0).astype(jnp.float32)

    @plsc.parallel_loop(0, ROWS, unroll=2)
    def _product_row(i):
        o = i * K
        acc = sibuf[pl.ds(o, 16)] * wbuf[pl.ds(o, 16)]
        acc = acc + sibuf[pl.ds(o + 16, 16)] * wbuf[pl.ds(o + 16, 16)]
        acc = acc + sibuf[pl.ds(o + 32, 16)] * wbuf[pl.ds(o + 32, 16)]
        acc = acc + sibuf[pl.ds(o + 36, 16)] * wbuf[pl.ds(o + 36, 16)] * tail_f
        plsc.addupdate_scatter(outv, [lanes * 0 + i], acc)

    bias_copy.wait()
    b = bias_v[...]
    for c in range(ROWS // 16):
        x = outv[pl.ds(c * 16, 16)] + b
        outv[pl.ds(c * 16, 16)] = 1.0 / (1.0 + jnp.exp(-x))

    pltpu.sync_copy(outv, out_hbm.at[pl.ds(wid * ROWS, ROWS)])


@jax.jit
def kernel(edges, W, bias):
    edges_flat = edges.reshape(-1)
    w_flat = W.reshape(-1)
    bias16 = jnp.broadcast_to(bias, (16,))
    pred = _build_singlenet_sc()(edges_flat, w_flat, bias16)
    return pred.reshape(B, 1)
